# Initial kernel scaffold; baseline (speedup 1.0000x reference)
import functools

import jax
import jax.numpy as jnp
from jax import lax
from jax.experimental import pallas as pl
from jax.experimental.pallas import tpu as pltpu
from jax.experimental.pallas import tpu_sc as plsc

N = 10000
E = 320000
D = 128
H = 128
O = 5

NC = 2
NS = 16
NW = NC * NS

C = 128
CHUNKS = -(-E // (NW * C))
E_PW = CHUNKS * C
E_PAD = E_PW * NW
N_PAD = 10240
ROWS_PS = N_PAD // NS


def _sc_mesh():
    return plsc.VectorSubcoreMesh(core_axis_name="c", subcore_axis_name="s")


def _sc_segment_sum(y, e_src, e_dst, with_cnt):
    out_types = [jax.ShapeDtypeStruct((NC, N_PAD, 128), jnp.float32)]
    if with_cnt:
        out_types.append(jax.ShapeDtypeStruct((NC, N_PAD, 16), jnp.float32))

    scratch = [
        pltpu.VMEM((C,), jnp.int32),
        pltpu.VMEM((C,), jnp.int32),
        pltpu.VMEM((C, 128), jnp.float32),
        pltpu.VMEM((C, 16), jnp.float32),
        pltpu.VMEM_SHARED((N_PAD, 128), jnp.float32),
        pltpu.VMEM_SHARED((N_PAD, 16), jnp.float32),
        pltpu.SemaphoreType.DMA,
    ]

    def body(y_hbm, src_hbm, dst_hbm, z128_hbm, z16_hbm, ones_hbm, *refs):
        if with_cnt:
            agg_out, cnt_out = refs[0], refs[1]
            scratch_refs = refs[2:]
        else:
            agg_out = refs[0]
            cnt_out = None
            scratch_refs = refs[1:]
        idx_s, idx_d, rows, ones_v, agg_sh, cnt_sh, sem = scratch_refs

        c = lax.axis_index("c")
        s = lax.axis_index("s")
        gw = c * NS + s

        pltpu.sync_copy(z128_hbm, agg_sh.at[pl.ds(s * ROWS_PS, ROWS_PS)])
        if with_cnt:
            pltpu.sync_copy(z16_hbm, cnt_sh.at[pl.ds(s * ROWS_PS, ROWS_PS)])
            pltpu.sync_copy(ones_hbm, ones_v)
        plsc.subcore_barrier()

        base = gw * E_PW

        @pl.loop(0, CHUNKS)
        def _(k):
            off = base + k * C
            pltpu.sync_copy(src_hbm.at[pl.ds(off, C)], idx_s)
            pltpu.sync_copy(dst_hbm.at[pl.ds(off, C)], idx_d)
            pltpu.async_copy(y_hbm.at[idx_s], rows, sem).wait()
            pltpu.sync_copy(rows, agg_sh.at[idx_d], add=True)
            if with_cnt:
                pltpu.sync_copy(ones_v, cnt_sh.at[idx_d], add=True)

        plsc.subcore_barrier()
        row0 = s * ROWS_PS
        pltpu.sync_copy(agg_sh.at[pl.ds(row0, ROWS_PS)],
                        agg_out.at[c].at[pl.ds(row0, ROWS_PS)])
        if with_cnt:
            pltpu.sync_copy(cnt_sh.at[pl.ds(row0, ROWS_PS)],
                            cnt_out.at[c].at[pl.ds(row0, ROWS_PS)])

    z128 = jnp.zeros((ROWS_PS, 128), jnp.float32)
    z16 = jnp.zeros((ROWS_PS, 16), jnp.float32)
    ones16 = jnp.ones((C, 16), jnp.float32)

    k = pl.kernel(body, out_type=out_types, mesh=_sc_mesh(),
                  scratch_types=scratch)
    return k(y, e_src, e_dst, z128, z16, ones16)


BLK = 1000


def _sage_tc_kernel(with_head, x_ref, a0, a1, c0, c1, wl, bl, wr, *rest):
    if with_head:
        w3, b3, o_ref = rest
    else:
        (o_ref,) = rest
    cnt = jnp.maximum(c0[:, :1] + c1[:, :1], 1.0)
    mean = (a0[...] + a1[...]) / cnt
    dn = (((1,), (1,)), ((), ()))
    t = lax.dot_general(mean, wl[...], dn, preferred_element_type=jnp.float32)
    r = lax.dot_general(x_ref[...], wr[...], dn,
                        preferred_element_type=jnp.float32)
    h = jnp.maximum(t + r + bl[...], 0.0)
    if with_head:
        o_ref[...] = lax.dot_general(
            h, w3[...], dn, preferred_element_type=jnp.float32) + b3[...]
    else:
        o_ref[...] = h


def _sage_tc(x, agg, cnt16, Wl, bl, Wr, head=None):
    row_spec = pl.BlockSpec((BLK, 128), lambda i: (i, 0))
    cnt_spec = pl.BlockSpec((BLK, 16), lambda i: (i, 0))
    w_spec = pl.BlockSpec((128, 128), lambda i: (0, 0))
    b_spec = pl.BlockSpec((1, 128), lambda i: (0, 0))
    in_specs = [row_spec, row_spec, row_spec, cnt_spec, cnt_spec,
                w_spec, b_spec, w_spec]
    args = [x, agg[0, :N], agg[1, :N], cnt16[0, :N], cnt16[1, :N],
            Wl, bl.reshape(1, 128), Wr]
    if head is not None:
        W3p, b3p = head
        in_specs += [w_spec, b_spec]
        args += [W3p, b3p]
    return pl.pallas_call(
        functools.partial(_sage_tc_kernel, head is not None),
        grid=(N // BLK,),
        in_specs=in_specs,
        out_specs=row_spec,
        out_shape=jax.ShapeDtypeStruct((N, 128), jnp.float32),
    )(*args)


def kernel(x, edge_index, W1l, b1l, W1r, W2l, b2l, W2r, W3, b3):
    src = edge_index[0]
    dst = edge_index[1]
    pad_i = jnp.arange(E_PAD - E, dtype=jnp.int32)
    e_src = jnp.concatenate([src, pad_i % N])
    e_dst = jnp.concatenate([dst, N + pad_i % (N_PAD - N)])

    agg1, cnt16 = _sc_segment_sum(x, e_src, e_dst, with_cnt=True)
    h = _sage_tc(x, agg1, cnt16, W1l, b1l, W1r)
    (agg2,) = _sc_segment_sum(h, e_src, e_dst, with_cnt=False)

    W3p = jnp.zeros((128, 128), jnp.float32).at[:O].set(W3)
    b3p = jnp.zeros((1, 128), jnp.float32).at[0, :O].set(b3)
    out = _sage_tc(h, agg2, cnt16, W2l, b2l, W2r, head=(W3p, b3p))
    return out[:, :O]


# speedup vs baseline: 5.6396x; 5.6396x over previous
import functools

import jax
import jax.numpy as jnp
from jax import lax
from jax.experimental import pallas as pl
from jax.experimental.pallas import tpu as pltpu
from jax.experimental.pallas import tpu_sc as plsc

N = 10000
E = 320000
D = 128
H = 128
O = 5

NC = 2
NS = 16
NW = NC * NS

C = 128
CHUNKS = -(-E // (NW * C))
E_PW = CHUNKS * C
E_PAD = E_PW * NW
N_PAD = 10240
ROWS_PS = N_PAD // NS


def _sc_mesh():
    return plsc.VectorSubcoreMesh(core_axis_name="c", subcore_axis_name="s",
                                  num_cores=NC, num_subcores=NS)


def _sc_segment_sum(y, e_src, e_dst):
    scratch = [
        pltpu.VMEM((1, C), jnp.int32),
        pltpu.VMEM((1, C), jnp.int32),
        pltpu.VMEM((C, 128), jnp.float32),
        pltpu.VMEM_SHARED((N_PAD, 128), jnp.float32),
        pltpu.SemaphoreType.DMA,
    ]

    def body(y_hbm, src_hbm, dst_hbm, z128_hbm, agg_out,
             idx_s, idx_d, rows, agg_sh, sem):
        c = lax.axis_index("c")
        s = lax.axis_index("s")
        gw = c * NS + s
        row0 = s * ROWS_PS

        pltpu.sync_copy(z128_hbm, rows)
        for j in range(ROWS_PS // C):
            pltpu.sync_copy(rows, agg_sh.at[pl.ds(row0 + j * C, C)])
        plsc.subcore_barrier()

        base = gw * E_PW

        @pl.loop(0, CHUNKS)
        def _(k):
            off = base + k * C
            pltpu.sync_copy(src_hbm.at[pl.ds(off, C)], idx_s.at[0])
            pltpu.sync_copy(dst_hbm.at[pl.ds(off, C)], idx_d.at[0])
            pltpu.async_copy(y_hbm.at[idx_s.at[0]], rows, sem).wait()
            pltpu.sync_copy(rows, agg_sh.at[idx_d.at[0]], add=True)

        plsc.subcore_barrier()
        for j in range(ROWS_PS // C):
            pltpu.sync_copy(agg_sh.at[pl.ds(row0 + j * C, C)], rows)
            pltpu.sync_copy(rows, agg_out.at[c].at[pl.ds(row0 + j * C, C)])

    z128 = jnp.zeros((C, 128), jnp.float32)
    k = pl.kernel(body,
                  out_type=jax.ShapeDtypeStruct((NC, N_PAD, 128), jnp.float32),
                  mesh=_sc_mesh(), scratch_types=scratch)
    return k(y, e_src, e_dst, z128)


def _sc_degree_count(e_dst):
    scratch = [
        pltpu.VMEM((1, C), jnp.int32),
        pltpu.VMEM((C, 128), jnp.float32),
        pltpu.VMEM_SHARED((N_PAD, 128), jnp.float32),
        pltpu.SemaphoreType.DMA,
    ]

    def body(dst_hbm, z16_hbm, ones_hbm, cnt_out, idx_d, ones_v, cnt_sh, sem):
        c = lax.axis_index("c")
        s = lax.axis_index("s")
        gw = c * NS + s
        row0 = s * ROWS_PS

        pltpu.sync_copy(z16_hbm, ones_v)
        for j in range(ROWS_PS // C):
            pltpu.sync_copy(ones_v, cnt_sh.at[pl.ds(row0 + j * C, C)])
        pltpu.sync_copy(ones_hbm, ones_v)
        plsc.subcore_barrier()

        base = gw * E_PW

        @pl.loop(0, CHUNKS)
        def _(k):
            pltpu.sync_copy(dst_hbm.at[pl.ds(base + k * C, C)], idx_d.at[0])
            pltpu.sync_copy(ones_v, cnt_sh.at[idx_d.at[0]], add=True)

        plsc.subcore_barrier()
        for j in range(ROWS_PS // C):
            pltpu.sync_copy(cnt_sh.at[pl.ds(row0 + j * C, C)], ones_v)
            pltpu.sync_copy(ones_v, cnt_out.at[c].at[pl.ds(row0 + j * C, C)])

    z16 = jnp.zeros((C, 128), jnp.float32)
    ones16 = jnp.ones((C, 128), jnp.float32)
    k = pl.kernel(body,
                  out_type=jax.ShapeDtypeStruct((NC, N_PAD, 128), jnp.float32),
                  mesh=_sc_mesh(), scratch_types=scratch)
    return k(e_dst, z16, ones16)


BLK = 1000


def _sage_tc_kernel(with_head, x_ref, a0, a1, c0, c1, wl, bl, wr, *rest):
    if with_head:
        w3, b3, o_ref = rest
    else:
        (o_ref,) = rest
    cnt = jnp.maximum(c0[:, :1] + c1[:, :1], 1.0)
    mean = (a0[...] + a1[...]) / cnt
    dn = (((1,), (1,)), ((), ()))
    t = lax.dot_general(mean, wl[...], dn, preferred_element_type=jnp.float32)
    r = lax.dot_general(x_ref[...], wr[...], dn,
                        preferred_element_type=jnp.float32)
    h = jnp.maximum(t + r + bl[...], 0.0)
    if with_head:
        o_ref[...] = lax.dot_general(
            h, w3[...], dn, preferred_element_type=jnp.float32) + b3[...]
    else:
        o_ref[...] = h


def _sage_tc(x, agg, cnt16, Wl, bl, Wr, head=None):
    row_spec = pl.BlockSpec((BLK, 128), lambda i: (i, 0))
    w_spec = pl.BlockSpec((128, 128), lambda i: (0, 0))
    b_spec = pl.BlockSpec((1, 128), lambda i: (0, 0))
    in_specs = [row_spec, row_spec, row_spec, row_spec, row_spec,
                w_spec, b_spec, w_spec]
    args = [x, agg[0, :N], agg[1, :N], cnt16[0, :N], cnt16[1, :N],
            Wl, bl.reshape(1, 128), Wr]
    if head is not None:
        W3p, b3p = head
        in_specs += [w_spec, b_spec]
        args += [W3p, b3p]
    return pl.pallas_call(
        functools.partial(_sage_tc_kernel, head is not None),
        grid=(N // BLK,),
        in_specs=in_specs,
        out_specs=row_spec,
        out_shape=jax.ShapeDtypeStruct((N, 128), jnp.float32),
    )(*args)


def kernel(x, edge_index, W1l, b1l, W1r, W2l, b2l, W2r, W3, b3):
    src = edge_index[0]
    dst = edge_index[1]
    pad_i = jnp.arange(E_PAD - E, dtype=jnp.int32)
    e_src = jnp.concatenate([src, pad_i % N])
    e_dst = jnp.concatenate([dst, N + pad_i % (N_PAD - N)])

    cnt16 = _sc_degree_count(e_dst)
    agg1 = _sc_segment_sum(x, e_src, e_dst)
    h = _sage_tc(x, agg1, cnt16, W1l, b1l, W1r)
    agg2 = _sc_segment_sum(h, e_src, e_dst)

    W3p = jnp.zeros((128, 128), jnp.float32).at[:O].set(W3)
    b3p = jnp.zeros((1, 128), jnp.float32).at[0, :O].set(b3)
    out = _sage_tc(h, agg2, cnt16, W2l, b2l, W2r, head=(W3p, b3p))
    return out[:, :O]


# speedup vs baseline: 7.8233x; 1.3872x over previous
import functools

import jax
import jax.numpy as jnp
from jax import lax
from jax.experimental import pallas as pl
from jax.experimental.pallas import tpu as pltpu
from jax.experimental.pallas import tpu_sc as plsc

N = 10000
E = 320000
D = 128
H = 128
O = 5

NC = 2
NS = 16
NW = NC * NS

C = 128
CHUNKS = 2 * (-(-E // (NW * C * 2)))
E_PW = CHUNKS * C
E_PAD = E_PW * NW
N_PAD = 10240
ROWS_PS = N_PAD // NS


def _sc_mesh():
    return plsc.VectorSubcoreMesh(core_axis_name="c", subcore_axis_name="s",
                                  num_cores=NC, num_subcores=NS)


def _sc_segment_sum(y, e_src, e_dst):
    scratch = [
        pltpu.VMEM((1, C), jnp.int32),
        pltpu.VMEM((1, C), jnp.int32),
        pltpu.VMEM((1, C), jnp.int32),
        pltpu.VMEM((1, C), jnp.int32),
        pltpu.VMEM((C, 128), jnp.float32),
        pltpu.VMEM((C, 128), jnp.float32),
        pltpu.VMEM_SHARED((N_PAD, 128), jnp.float32),
        pltpu.SemaphoreType.DMA,
        pltpu.SemaphoreType.DMA,
    ]

    def body(y_hbm, src_hbm, dst_hbm, z128_hbm, agg_out,
             sa, sb, da, db, rows_a, rows_b, agg_sh, sem_a, sem_b):
        c = lax.axis_index("c")
        s = lax.axis_index("s")
        gw = c * NS + s
        row0 = s * ROWS_PS
        base = gw * E_PW

        pltpu.sync_copy(z128_hbm, rows_a)
        for j in range(ROWS_PS // C):
            pltpu.sync_copy(rows_a, agg_sh.at[pl.ds(row0 + j * C, C)])
        plsc.subcore_barrier()

        pltpu.sync_copy(src_hbm.at[pl.ds(base, C)], sa.at[0])
        pltpu.async_copy(y_hbm.at[sa.at[0]], rows_a, sem_a)

        @pl.loop(0, CHUNKS // 2)
        def _(i):
            k0 = 2 * i
            pltpu.sync_copy(src_hbm.at[pl.ds(base + (k0 + 1) * C, C)],
                            sb.at[0])
            pltpu.sync_copy(dst_hbm.at[pl.ds(base + k0 * C, C)], da.at[0])
            pltpu.make_async_copy(y_hbm.at[sa.at[0]], rows_a, sem_a).wait()
            pltpu.async_copy(y_hbm.at[sb.at[0]], rows_b, sem_b)
            pltpu.sync_copy(rows_a, agg_sh.at[da.at[0]], add=True)
            pltpu.sync_copy(dst_hbm.at[pl.ds(base + (k0 + 1) * C, C)],
                            db.at[0])

            @pl.when(k0 + 2 < CHUNKS)
            def _():
                pltpu.sync_copy(src_hbm.at[pl.ds(base + (k0 + 2) * C, C)],
                                sa.at[0])

            pltpu.make_async_copy(y_hbm.at[sb.at[0]], rows_b, sem_b).wait()

            @pl.when(k0 + 2 < CHUNKS)
            def _():
                pltpu.async_copy(y_hbm.at[sa.at[0]], rows_a, sem_a)

            pltpu.sync_copy(rows_b, agg_sh.at[db.at[0]], add=True)

        plsc.subcore_barrier()
        for j in range(ROWS_PS // C):
            buf = rows_a if j % 2 == 0 else rows_b
            pltpu.sync_copy(agg_sh.at[pl.ds(row0 + j * C, C)], buf)
            pltpu.sync_copy(buf, agg_out.at[c].at[pl.ds(row0 + j * C, C)])

    z128 = jnp.zeros((C, 128), jnp.float32)
    k = pl.kernel(body,
                  out_type=jax.ShapeDtypeStruct((NC, N_PAD, 128), jnp.float32),
                  mesh=_sc_mesh(), scratch_types=scratch)
    return k(y, e_src, e_dst, z128)


def _sc_degree_count(e_dst):
    scratch = [
        pltpu.VMEM((1, C), jnp.int32),
        pltpu.VMEM((C, 128), jnp.float32),
        pltpu.VMEM_SHARED((N_PAD, 128), jnp.float32),
        pltpu.SemaphoreType.DMA,
    ]

    def body(dst_hbm, z16_hbm, ones_hbm, cnt_out, dv, ones_v, cnt_sh, sem):
        c = lax.axis_index("c")
        s = lax.axis_index("s")
        gw = c * NS + s
        row0 = s * ROWS_PS
        base = gw * E_PW

        pltpu.sync_copy(z16_hbm, ones_v)
        for j in range(ROWS_PS // C):
            pltpu.sync_copy(ones_v, cnt_sh.at[pl.ds(row0 + j * C, C)])
        pltpu.sync_copy(ones_hbm, ones_v)
        plsc.subcore_barrier()

        @pl.loop(0, CHUNKS)
        def _(k):
            pltpu.sync_copy(dst_hbm.at[pl.ds(base + k * C, C)], dv.at[0])
            pltpu.sync_copy(ones_v, cnt_sh.at[dv.at[0]], add=True)

        plsc.subcore_barrier()
        for j in range(ROWS_PS // C):
            pltpu.sync_copy(cnt_sh.at[pl.ds(row0 + j * C, C)], ones_v)
            pltpu.sync_copy(ones_v, cnt_out.at[c].at[pl.ds(row0 + j * C, C)])

    z16 = jnp.zeros((C, 128), jnp.float32)
    ones16 = jnp.ones((C, 128), jnp.float32)
    k = pl.kernel(body,
                  out_type=jax.ShapeDtypeStruct((NC, N_PAD, 128), jnp.float32),
                  mesh=_sc_mesh(), scratch_types=scratch)
    return k(e_dst, z16, ones16)


BLK = 1000


def _sage_tc_kernel(with_head, x_ref, a0, a1, c0, c1, wl, bl, wr, *rest):
    if with_head:
        w3, b3, o_ref = rest
    else:
        (o_ref,) = rest
    cnt = jnp.maximum(c0[:, :1] + c1[:, :1], 1.0)
    mean = (a0[...] + a1[...]) / cnt
    dn = (((1,), (1,)), ((), ()))
    t = lax.dot_general(mean, wl[...], dn, preferred_element_type=jnp.float32)
    r = lax.dot_general(x_ref[...], wr[...], dn,
                        preferred_element_type=jnp.float32)
    h = jnp.maximum(t + r + bl[...], 0.0)
    if with_head:
        o_ref[...] = lax.dot_general(
            h, w3[...], dn, preferred_element_type=jnp.float32) + b3[...]
    else:
        o_ref[...] = h


def _sage_tc(x, agg, cnt16, Wl, bl, Wr, head=None):
    row_spec = pl.BlockSpec((BLK, 128), lambda i: (i, 0))
    w_spec = pl.BlockSpec((128, 128), lambda i: (0, 0))
    b_spec = pl.BlockSpec((1, 128), lambda i: (0, 0))
    in_specs = [row_spec, row_spec, row_spec, row_spec, row_spec,
                w_spec, b_spec, w_spec]
    args = [x, agg[0, :N], agg[1, :N], cnt16[0, :N], cnt16[1, :N],
            Wl, bl.reshape(1, 128), Wr]
    if head is not None:
        W3p, b3p = head
        in_specs += [w_spec, b_spec]
        args += [W3p, b3p]
    return pl.pallas_call(
        functools.partial(_sage_tc_kernel, head is not None),
        grid=(N // BLK,),
        in_specs=in_specs,
        out_specs=row_spec,
        out_shape=jax.ShapeDtypeStruct((N, 128), jnp.float32),
    )(*args)


def kernel(x, edge_index, W1l, b1l, W1r, W2l, b2l, W2r, W3, b3):
    src = edge_index[0]
    dst = edge_index[1]
    pad_i = jnp.arange(E_PAD - E, dtype=jnp.int32)
    e_src = jnp.concatenate([src, pad_i % N])
    e_dst = jnp.concatenate([dst, N + pad_i % (N_PAD - N)])

    cnt16 = _sc_degree_count(e_dst)
    agg1 = _sc_segment_sum(x, e_src, e_dst)
    h = _sage_tc(x, agg1, cnt16, W1l, b1l, W1r)
    agg2 = _sc_segment_sum(h, e_src, e_dst)

    W3p = jnp.zeros((128, 128), jnp.float32).at[:O].set(W3)
    b3p = jnp.zeros((1, 128), jnp.float32).at[0, :O].set(b3)
    out = _sage_tc(h, agg2, cnt16, W2l, b2l, W2r, head=(W3p, b3p))
    return out[:, :O]
